# baseline (speedup 1.0000x reference)
import functools

import jax
import jax.numpy as jnp
from jax import lax
from jax.experimental import pallas as pl
from jax.experimental.pallas import tpu as pltpu
from jax.experimental.pallas import tpu_sc as plsc

L = 16
EPS = 1e-5


_GDN = lax.GatherDimensionNumbers(
    offset_dims=(), collapsed_slice_dims=(0,), start_index_map=(0,))


def _perm(v, idx):
    return lax.gather(v, idx[:, None], dimension_numbers=_GDN,
                      slice_sizes=(1,),
                      mode=lax.GatherScatterMode.PROMISE_IN_BOUNDS)


def _splat_lane(v, k):
    return _perm(v, jnp.full((L,), k, jnp.int32))


def _vsum(v):
    for s in (1, 2, 4, 8):
        v = v + _perm(v, jnp.arange(L, dtype=jnp.int32) ^ s)
    return v


def _rsqrt(x):
    i = lax.bitcast_convert_type(x, jnp.int32)
    y = lax.bitcast_convert_type(jnp.int32(0x5F3759DF) - (i >> 1), jnp.float32)
    for _ in range(3):
        y = y * (1.5 - 0.5 * x * y * y)
    return y


def _sc_info():
    try:
        info = plsc.get_sparse_core_info()
        return info.num_cores, info.num_subcores
    except RuntimeError:
        return 2, 16


def _make_sc_kernel(V, B, NOCR, S, H, n_pad, rpw, W):
    NC, NS = _sc_info()
    HC = H // L
    general_type = S > V
    mesh = plsc.VectorSubcoreMesh(core_axis_name="c", subcore_axis_name="s")

    @functools.partial(
        pl.kernel,
        mesh=mesh,
        compiler_params=pltpu.CompilerParams(needs_layout_passes=False),
        out_type=jax.ShapeDtypeStruct((n_pad, H), jnp.float32),
        scratch_types=[
            pltpu.VMEM((W,), jnp.int32),
            pltpu.VMEM((W,), jnp.int32),
            pltpu.VMEM((W,), jnp.int32),
            pltpu.VMEM((W,), jnp.int32),
            pltpu.VMEM((W,), jnp.float32),
            pltpu.VMEM((rpw, H), jnp.float32),
            pltpu.VMEM((rpw, H), jnp.float32),
            pltpu.VMEM((rpw, H), jnp.float32),
            pltpu.VMEM((8, H), jnp.float32),
            pltpu.VMEM((rpw, H), jnp.float32),
            pltpu.VMEM((rpw,), jnp.int32),
            pltpu.SemaphoreType.DMA,
            pltpu.SemaphoreType.DMA,
            pltpu.SemaphoreType.DMA,
            pltpu.SemaphoreType.DMA,
        ],
    )
    def sc_kernel(cv_hbm, ocr_hbm, ids_hbm, pos_hbm, params_hbm,
                  out_hbm, idx_win, cidx, oidx, pidx, rflag,
                  crow, orow, prow, params_v, outbuf, widx,
                  sem0, sem1, sem2, sem3):
        wid = lax.axis_index("s") * NC + lax.axis_index("c")
        base = wid * rpw
        abase = pl.multiple_of(jnp.minimum(base - lax.rem(base, 8), n_pad - W), 8)
        pltpu.sync_copy(ids_hbm.at[pl.ds(abase, W)], idx_win)

        for j in range(W // L):
            v = idx_win[pl.ds(j * L, L)]
            p = abase + j * L + lax.iota(jnp.int32, L)
            r = p - base
            msk = (r >= 0) & (r < rpw)
            b = lax.div(p, S)
            is_ocr = v >= V
            plsc.store_scatter(cidx, [r], jnp.where(is_ocr, lax.rem(p, V), v),
                               mask=msk)
            plsc.store_scatter(oidx, [r],
                               jnp.where(is_ocr, b * NOCR + (v - V),
                                         lax.rem(p, B * NOCR)), mask=msk)
            plsc.store_scatter(pidx, [r], lax.rem(p, S), mask=msk)
            plsc.store_scatter(rflag, [r], jnp.where(is_ocr, 1.0, 0.0),
                               mask=msk)
            plsc.store_scatter(widx, [r], p, mask=msk)

        d0 = pltpu.async_copy(cv_hbm.at[cidx.at[pl.ds(0, rpw)]], crow, sem0)
        d1 = pltpu.async_copy(ocr_hbm.at[oidx.at[pl.ds(0, rpw)]], orow, sem1)
        d2 = pltpu.async_copy(pos_hbm.at[pidx.at[pl.ds(0, rpw)]], prow, sem2)
        pltpu.sync_copy(params_hbm, params_v)
        d0.wait(); d1.wait(); d2.wait()

        tt0 = [params_v[6, pl.ds(j * L, L)] for j in range(HC)]
        tt1 = ([params_v[7, pl.ds(j * L, L)] for j in range(HC)]
               if general_type else None)

        @plsc.parallel_loop(0, rpw, 1, unroll=8)
        def row(i):
            fv = rflag[pl.ds((i // L) * L, L)]
            t = _splat_lane(fv, i % L)
            if general_type:
                sv = pidx[pl.ds((i // L) * L, L)]
                t2 = jnp.where(_splat_lane(sv, i % L) >= V, 1.0, 0.0)
            xs, ps = [], []
            acc1 = jnp.zeros((L,), jnp.float32)
            acc2 = jnp.zeros((L,), jnp.float32)
            pacc1 = jnp.zeros((L,), jnp.float32)
            pacc2 = jnp.zeros((L,), jnp.float32)
            for j in range(HC):
                c = crow[i, pl.ds(j * L, L)]
                o = orow[i, pl.ds(j * L, L)]
                x = c + t * (o - c)
                xs.append(x)
                acc1 += x
                acc2 += x * x
                ttj = (tt0[j] if not general_type
                       else tt0[j] + t2 * (tt1[j] - tt0[j]))
                pt = prow[i, pl.ds(j * L, L)] + ttj
                ps.append(pt)
                pacc1 += pt
                pacc2 += pt * pt
            mv = _vsum(acc1) * (1.0 / H)
            invv = _rsqrt(_vsum(acc2) * (1.0 / H) - mv * mv + EPS)
            pmv = _vsum(pacc1) * (1.0 / H)
            pinvv = _rsqrt(_vsum(pacc2) * (1.0 / H) - pmv * pmv + EPS)
            for j in range(HC):
                w = (params_v[0, pl.ds(j * L, L)]
                     + t * params_v[2, pl.ds(j * L, L)])
                bb = (params_v[1, pl.ds(j * L, L)]
                      + t * params_v[3, pl.ds(j * L, L)])
                y = (xs[j] - mv) * invv * w + bb
                y2 = ((ps[j] - pmv) * pinvv * params_v[4, pl.ds(j * L, L)]
                      + params_v[5, pl.ds(j * L, L)])
                outbuf[i, pl.ds(j * L, L)] = y + y2

        pltpu.async_copy(outbuf, out_hbm.at[widx], sem3).wait()

    return sc_kernel


def kernel(common_voc_embed, ocr_embed, prev_ids, pos_table, type_table,
           cv_ln_w, cv_ln_b, ocr_ln_w, ocr_ln_b, emb_ln_w, emb_ln_b):
    V, H = common_voc_embed.shape
    B, NOCR, _ = ocr_embed.shape
    _, S = prev_ids.shape
    NC, NS = _sc_info()
    NW = NC * NS
    n = B * S
    rpw = -(-n // NW)
    W = -(-(rpw + 7) // L) * L
    n_pad = NW * rpw

    ids = prev_ids.reshape(n)
    if n_pad != n:
        ids = jnp.concatenate([ids, jnp.zeros((n_pad - n,), jnp.int32)])
    ocr_flat = ocr_embed.reshape(B * NOCR, H)
    params = jnp.concatenate([
        jnp.stack([cv_ln_w, cv_ln_b, ocr_ln_w - cv_ln_w,
                   ocr_ln_b - cv_ln_b, emb_ln_w, emb_ln_b]), type_table])

    sc = _make_sc_kernel(V, B, NOCR, S, H, n_pad, rpw, W)
    out = sc(common_voc_embed, ocr_flat, ids, pos_table, params)
    if n_pad != n:
        out = out[:n]
    return out.reshape(B, S, H)


# speedup vs baseline: 1.0265x; 1.0265x over previous
import functools

import jax
import jax.numpy as jnp
from jax import lax
from jax.experimental import pallas as pl
from jax.experimental.pallas import tpu as pltpu
from jax.experimental.pallas import tpu_sc as plsc

L = 16
EPS = 1e-5


_GDN = lax.GatherDimensionNumbers(
    offset_dims=(), collapsed_slice_dims=(0,), start_index_map=(0,))


def _perm(v, idx):
    return lax.gather(v, idx[:, None], dimension_numbers=_GDN,
                      slice_sizes=(1,),
                      mode=lax.GatherScatterMode.PROMISE_IN_BOUNDS)


def _splat_lane(v, k):
    return _perm(v, jnp.full((L,), k, jnp.int32))


def _vsum(v):
    for s in (1, 2, 4, 8):
        v = v + _perm(v, jnp.arange(L, dtype=jnp.int32) ^ s)
    return v


def _rsqrt(x):
    i = lax.bitcast_convert_type(x, jnp.int32)
    y = lax.bitcast_convert_type(jnp.int32(0x5F3759DF) - (i >> 1), jnp.float32)
    for _ in range(3):
        y = y * (1.5 - 0.5 * x * y * y)
    return y


def _sc_info():
    try:
        info = plsc.get_sparse_core_info()
        return info.num_cores, info.num_subcores
    except RuntimeError:
        return 2, 16


def _make_sc_kernel(V, B, NOCR, S, H, n_pad, rpw, W):
    NC, NS = _sc_info()
    HC = H // L
    general_type = S > V
    mesh = plsc.VectorSubcoreMesh(core_axis_name="c", subcore_axis_name="s")

    @functools.partial(
        pl.kernel,
        mesh=mesh,
        compiler_params=pltpu.CompilerParams(needs_layout_passes=False),
        out_type=jax.ShapeDtypeStruct((n_pad, H), jnp.float32),
        scratch_types=[
            pltpu.VMEM((W,), jnp.int32),
            pltpu.VMEM((W,), jnp.int32),
            pltpu.VMEM((W,), jnp.int32),
            pltpu.VMEM((W,), jnp.int32),
            pltpu.VMEM((W,), jnp.float32),
            pltpu.VMEM((rpw, H), jnp.float32),
            pltpu.VMEM((rpw, H), jnp.float32),
            pltpu.VMEM((rpw, H), jnp.float32),
            pltpu.VMEM((8, H), jnp.float32),
            pltpu.VMEM((rpw, H), jnp.float32),
            pltpu.VMEM((rpw,), jnp.int32),
            pltpu.SemaphoreType.DMA,
            pltpu.SemaphoreType.DMA,
            pltpu.SemaphoreType.DMA,
            pltpu.SemaphoreType.DMA,
        ],
    )
    def sc_kernel(cv_hbm, ocr_hbm, ids_hbm, pos_hbm, params_hbm,
                  out_hbm, idx_win, cidx, oidx, pidx, rflag,
                  crow, orow, prow, params_v, outbuf, widx,
                  sem0, sem1, sem2, sem3):
        wid = lax.axis_index("s") * NC + lax.axis_index("c")
        base = wid * rpw
        abase = pl.multiple_of(jnp.minimum(base - lax.rem(base, 8), n_pad - W), 8)
        pltpu.sync_copy(ids_hbm.at[pl.ds(abase, W)], idx_win)

        for j in range(W // L):
            v = idx_win[pl.ds(j * L, L)]
            p = abase + j * L + lax.iota(jnp.int32, L)
            r = p - base
            msk = (r >= 0) & (r < rpw)
            b = lax.div(p, S)
            is_ocr = v >= V
            plsc.store_scatter(cidx, [r], jnp.where(is_ocr, lax.rem(p, V), v),
                               mask=msk)
            plsc.store_scatter(oidx, [r],
                               jnp.where(is_ocr, b * NOCR + (v - V),
                                         lax.rem(p, B * NOCR)), mask=msk)
            plsc.store_scatter(pidx, [r], lax.rem(p, S), mask=msk)
            plsc.store_scatter(rflag, [r], jnp.where(is_ocr, 1.0, 0.0),
                               mask=msk)
            plsc.store_scatter(widx, [r], p, mask=msk)

        d0 = pltpu.async_copy(cv_hbm.at[cidx.at[pl.ds(0, rpw)]], crow, sem0)
        d1 = pltpu.async_copy(ocr_hbm.at[oidx.at[pl.ds(0, rpw)]], orow, sem1)
        d2 = pltpu.async_copy(pos_hbm.at[pidx.at[pl.ds(0, rpw)]], prow, sem2)
        pltpu.sync_copy(params_hbm, params_v)
        d0.wait(); d1.wait(); d2.wait()

        tt0 = [params_v[6, pl.ds(j * L, L)] for j in range(HC)]
        tt1 = ([params_v[7, pl.ds(j * L, L)] for j in range(HC)]
               if general_type else None)

        @plsc.parallel_loop(0, rpw, 1, unroll=5)
        def row(i):
            fv = rflag[pl.ds((i // L) * L, L)]
            t = _splat_lane(fv, i % L)
            if general_type:
                sv = pidx[pl.ds((i // L) * L, L)]
                t2 = jnp.where(_splat_lane(sv, i % L) >= V, 1.0, 0.0)
            xs, ps = [], []
            acc1 = jnp.zeros((L,), jnp.float32)
            acc2 = jnp.zeros((L,), jnp.float32)
            pacc1 = jnp.zeros((L,), jnp.float32)
            pacc2 = jnp.zeros((L,), jnp.float32)
            for j in range(HC):
                c = crow[i, pl.ds(j * L, L)]
                o = orow[i, pl.ds(j * L, L)]
                x = c + t * (o - c)
                xs.append(x)
                acc1 += x
                acc2 += x * x
                ttj = (tt0[j] if not general_type
                       else tt0[j] + t2 * (tt1[j] - tt0[j]))
                pt = prow[i, pl.ds(j * L, L)] + ttj
                ps.append(pt)
                pacc1 += pt
                pacc2 += pt * pt
            mv = _vsum(acc1) * (1.0 / H)
            invv = _rsqrt(_vsum(acc2) * (1.0 / H) - mv * mv + EPS)
            pmv = _vsum(pacc1) * (1.0 / H)
            pinvv = _rsqrt(_vsum(pacc2) * (1.0 / H) - pmv * pmv + EPS)
            for j in range(HC):
                w = (params_v[0, pl.ds(j * L, L)]
                     + t * params_v[2, pl.ds(j * L, L)])
                bb = (params_v[1, pl.ds(j * L, L)]
                      + t * params_v[3, pl.ds(j * L, L)])
                y = (xs[j] - mv) * invv * w + bb
                y2 = ((ps[j] - pmv) * pinvv * params_v[4, pl.ds(j * L, L)]
                      + params_v[5, pl.ds(j * L, L)])
                outbuf[i, pl.ds(j * L, L)] = y + y2

        pltpu.async_copy(outbuf, out_hbm.at[widx], sem3).wait()

    return sc_kernel


def kernel(common_voc_embed, ocr_embed, prev_ids, pos_table, type_table,
           cv_ln_w, cv_ln_b, ocr_ln_w, ocr_ln_b, emb_ln_w, emb_ln_b):
    V, H = common_voc_embed.shape
    B, NOCR, _ = ocr_embed.shape
    _, S = prev_ids.shape
    NC, NS = _sc_info()
    NW = NC * NS
    n = B * S
    rpw = -(-n // NW)
    W = -(-(rpw + 7) // L) * L
    n_pad = NW * rpw

    ids = prev_ids.reshape(n)
    if n_pad != n:
        ids = jnp.concatenate([ids, jnp.zeros((n_pad - n,), jnp.int32)])
    ocr_flat = ocr_embed.reshape(B * NOCR, H)
    params = jnp.concatenate([
        jnp.stack([cv_ln_w, cv_ln_b, ocr_ln_w - cv_ln_w,
                   ocr_ln_b - cv_ln_b, emb_ln_w, emb_ln_b]), type_table])

    sc = _make_sc_kernel(V, B, NOCR, S, H, n_pad, rpw, W)
    out = sc(common_voc_embed, ocr_flat, ids, pos_table, params)
    if n_pad != n:
        out = out[:n]
    return out.reshape(B, S, H)
